# baseline (speedup 1.0000x reference)
import functools

import jax
import jax.numpy as jnp
from jax import lax
from jax.experimental import pallas as pl
from jax.experimental.pallas import tpu as pltpu
from jax.experimental.pallas import tpu_sc as plsc

N = 10000
NP = 10240
E = 320000
F = 128
OUT_F = 64

NC = 2
NS = 16
L = 16
NW = NC * NS
EPT = E // NW
EPS = E // NS
RPT = NP // NS

EB = 128
NB = 80
CB = 40
NCH = NB // CB
EPAD = NW * NB * EB
ZR = 128

_SC_PARAMS = pltpu.CompilerParams(needs_layout_passes=False, use_tc_tiling_on_sc=False)

_f32 = jnp.float32
_i32 = jnp.int32


def _mesh():
    return plsc.VectorSubcoreMesh(
        core_axis_name="c", subcore_axis_name="s",
        num_cores=NC, num_subcores=NS)


def _rsqrt16(d):
    bits = plsc.bitcast(d, _i32)
    y = plsc.bitcast(jnp.int32(0x5F3759DF) - (bits >> 1), _f32)
    for _ in range(3):
        y = y * (1.5 - 0.5 * d * y * y)
    return jnp.where(d > 0.0, y, 0.0)


@functools.partial(
    pl.kernel,
    out_type=jax.ShapeDtypeStruct((NW, NB, EB), _f32),
    mesh=_mesh(),
    scratch_types=[
        pltpu.VMEM_SHARED((NS, NP), _f32),
        pltpu.VMEM_SHARED((NP,), _f32),
        pltpu.VMEM((NP,), _f32),
        pltpu.VMEM((NP,), _f32),
        pltpu.VMEM((NS, RPT), _f32),
        pltpu.VMEM((NB, EB), _i32),
        pltpu.VMEM((NB, EB), _i32),
        pltpu.VMEM((NB, EB), _f32),
        pltpu.VMEM((NB, EB), _f32),
    ],
    compiler_params=_SC_PARAMS,
)
def _wnorm(src_hbm, dst_hbm, ew_hbm, wn_hbm,
           deg_sh, dinv_sh, deg_v, dinv_v, sum_v, sv_v, dv_v, ev_v, wn_v):
    c = lax.axis_index("c")
    s = lax.axis_index("s")
    wid = c * NS + s

    def zero_deg(i, _):
        deg_v[pl.ds(i * L, L)] = jnp.zeros((L,), _f32)
        return 0
    lax.fori_loop(0, NP // L, zero_deg, 0)

    for half in range(2):
        chunk = s * 2 + half
        pltpu.sync_copy(src_hbm.at[chunk], sv_v)
        pltpu.sync_copy(ew_hbm.at[chunk], ev_v)

        def deg_batch(g, _):
            def inner(j, _):
                idx = sv_v[g, pl.ds(j * L, L)]
                val = ev_v[g, pl.ds(j * L, L)]
                plsc.addupdate_scatter(deg_v, [idx], val)
                return 0
            lax.fori_loop(0, EB // L, inner, 0)
            return 0
        lax.fori_loop(0, NB, deg_batch, 0)

    pltpu.sync_copy(deg_v, deg_sh.at[s])
    plsc.subcore_barrier()

    pltpu.sync_copy(deg_sh.at[:, pl.ds(s * RPT, RPT)], sum_v)

    def dinv_blk(j, _):
        d = jnp.zeros((L,), _f32)
        for r in range(NS):
            d = d + sum_v[r, pl.ds(j * L, L)]
        dinv_v[pl.ds(s * RPT + j * L, L)] = _rsqrt16(d)
        return 0
    lax.fori_loop(0, RPT // L, dinv_blk, 0)

    pltpu.sync_copy(dinv_v.at[pl.ds(s * RPT, RPT)],
                    dinv_sh.at[pl.ds(s * RPT, RPT)])
    plsc.subcore_barrier()
    pltpu.sync_copy(dinv_sh, dinv_v)

    pltpu.sync_copy(src_hbm.at[wid], sv_v)
    pltpu.sync_copy(dst_hbm.at[wid], dv_v)
    pltpu.sync_copy(ew_hbm.at[wid], ev_v)

    def wn_batch(g, _):
        def inner(j, _):
            si = sv_v[g, pl.ds(j * L, L)]
            di = dv_v[g, pl.ds(j * L, L)]
            ew = ev_v[g, pl.ds(j * L, L)]
            a = plsc.load_gather(dinv_v, [si])
            b = plsc.load_gather(dinv_v, [di])
            wn_v[g, pl.ds(j * L, L)] = -(a * ew * b)
            return 0
        lax.fori_loop(0, EB // L, inner, 0)
        return 0
    lax.fori_loop(0, NB, wn_batch, 0)
    pltpu.sync_copy(wn_v, wn_hbm.at[wid])


@functools.partial(
    pl.kernel,
    out_type=jax.ShapeDtypeStruct((NC, NP, F), _f32),
    mesh=_mesh(),
    scratch_types=[
        pltpu.VMEM_SHARED((NP, F), _f32),
        pltpu.VMEM((CB, EB), _i32),
        pltpu.VMEM((CB, EB), _i32),
        pltpu.VMEM((CB, EB), _f32),
        pltpu.VMEM((EB, F), jnp.bfloat16),
        pltpu.VMEM((EB, F), jnp.bfloat16),
        pltpu.VMEM((EB, F), _f32),
        pltpu.SemaphoreType.DMA,
        pltpu.SemaphoreType.DMA,
        pltpu.SemaphoreType.DMA,
        pltpu.SemaphoreType.DMA,
    ],
    compiler_params=_SC_PARAMS,
)
def _lhat(v_hbm, src_hbm, dst_hbm, wn_hbm, p_hbm,
          acc_sh, sv_v, dv_v, wv_v, rows0, rows1, rows_out, sg0, sg1, ss0, ss1):
    c = lax.axis_index("c")
    s = lax.axis_index("s")
    wid = c * NS + s

    def zero_st(i, _):
        rows_out[i // (F // L), pl.ds((i % (F // L)) * L, L)] = jnp.zeros((L,), _f32)
        return 0
    lax.fori_loop(0, ZR * (F // L), zero_st, 0)
    for t in range(RPT // ZR):
        pltpu.sync_copy(rows_out, acc_sh.at[pl.ds(s * RPT + t * ZR, ZR)])
    plsc.subcore_barrier()

    def scale(rows, lg):
        def sc(j, _):
            w16 = wv_v[lg, pl.ds(j * L, L)]
            for bb in range(L):
                b = j * L + bb
                w = w16[bb]
                for fg in range(F // 32):
                    xb = rows[b, pl.ds(fg * 32, 32)]
                    lo, hi = plsc.unpack(xb, format=plsc.PackFormat.INTERLEAVED)
                    rows_out[b, pl.ds(fg * 32, L)] = lo * w
                    rows_out[b, pl.ds(fg * 32 + L, L)] = hi * w
            return 0
        lax.fori_loop(0, EB // L, sc, 0)

    for k in range(NCH):
        pltpu.sync_copy(src_hbm.at[wid, k], sv_v)
        pltpu.sync_copy(dst_hbm.at[wid, k], dv_v)
        pltpu.sync_copy(wn_hbm.at[wid, k], wv_v)

        pltpu.async_copy(v_hbm.at[sv_v.at[0]], rows0, sg0)
        pltpu.async_copy(v_hbm.at[sv_v.at[1]], rows1, sg1)

        def pipe(gg, _):
            g0 = 2 * gg
            g1 = 2 * gg + 1
            pltpu.make_async_copy(v_hbm.at[sv_v.at[g0]], rows0, sg0).wait()
            scale(rows0, g0)
            pltpu.sync_copy(rows_out, acc_sh.at[dv_v.at[g0]], add=True)
            pltpu.async_copy(v_hbm.at[sv_v.at[g0 + 2]], rows0, sg0)
            pltpu.make_async_copy(v_hbm.at[sv_v.at[g1]], rows1, sg1).wait()
            scale(rows1, g1)
            pltpu.sync_copy(rows_out, acc_sh.at[dv_v.at[g1]], add=True)
            pltpu.async_copy(v_hbm.at[sv_v.at[g1 + 2]], rows1, sg1)
            return 0
        lax.fori_loop(0, CB // 2 - 1, pipe, 0)

        pltpu.make_async_copy(v_hbm.at[sv_v.at[CB - 2]], rows0, sg0).wait()
        scale(rows0, CB - 2)
        pltpu.sync_copy(rows_out, acc_sh.at[dv_v.at[CB - 2]], add=True)
        pltpu.make_async_copy(v_hbm.at[sv_v.at[CB - 1]], rows1, sg1).wait()
        scale(rows1, CB - 1)
        pltpu.sync_copy(rows_out, acc_sh.at[dv_v.at[CB - 1]], add=True)

    plsc.subcore_barrier()
    for t in range(RPT // ZR):
        pltpu.sync_copy(acc_sh.at[pl.ds(s * RPT + t * ZR, ZR)],
                        p_hbm.at[c, pl.ds(s * RPT + t * ZR, ZR)])


BN_ROWS = 256
GRID = NP // BN_ROWS


def _bf16_pack(v):
    vi = v.reshape(BN_ROWS, 4, 2, 16).transpose(0, 1, 3, 2).reshape(BN_ROWS, F)
    return vi.astype(jnp.bfloat16)


def _tc_add2_body(p0_ref, p1_ref, tx1_ref, txbf_ref):
    t = p0_ref[0] + p1_ref[0]
    tx1_ref[...] = t
    txbf_ref[...] = _bf16_pack(t)


def _tc_add2(p):
    return pl.pallas_call(
        _tc_add2_body,
        grid=(GRID,),
        in_specs=[
            pl.BlockSpec((1, BN_ROWS, F), lambda i: (0, i, 0)),
            pl.BlockSpec((1, BN_ROWS, F), lambda i: (1, i, 0)),
        ],
        out_specs=(pl.BlockSpec((BN_ROWS, F), lambda i: (i, 0)),
                   pl.BlockSpec((BN_ROWS, F), lambda i: (i, 0))),
        out_shape=(jax.ShapeDtypeStruct((NP, F), _f32),
                   jax.ShapeDtypeStruct((NP, F), jnp.bfloat16)),
    )(p, p)


def _tc_pack_body(v_ref, o_ref):
    o_ref[...] = _bf16_pack(v_ref[...])


def _tc_pack(v):
    return pl.pallas_call(
        _tc_pack_body,
        grid=(GRID,),
        in_specs=[pl.BlockSpec((BN_ROWS, F), lambda i: (i, 0))],
        out_specs=pl.BlockSpec((BN_ROWS, F), lambda i: (i, 0)),
        out_shape=jax.ShapeDtypeStruct((NP, F), jnp.bfloat16),
    )(v)


def _tc_part_body(v_ref, tx1_ref, w0_ref, w1_ref, part_ref):
    part_ref[...] = (
        jnp.dot(v_ref[...], w0_ref[...], preferred_element_type=_f32)
        + jnp.dot(tx1_ref[...], w1_ref[...], preferred_element_type=_f32))


def _tc_part(v, tx1, w0, w1):
    return pl.pallas_call(
        _tc_part_body,
        grid=(GRID,),
        in_specs=[
            pl.BlockSpec((BN_ROWS, F), lambda i: (i, 0)),
            pl.BlockSpec((BN_ROWS, F), lambda i: (i, 0)),
            pl.BlockSpec((F, F), lambda i: (0, 0)),
            pl.BlockSpec((F, F), lambda i: (0, 0)),
        ],
        out_specs=pl.BlockSpec((BN_ROWS, F), lambda i: (i, 0)),
        out_shape=jax.ShapeDtypeStruct((NP, F), _f32),
    )(v, tx1, w0, w1)


def _tc_layer1_body(q0_ref, q1_ref, v_ref, part_ref, w2_ref, b_ref,
                    gam_ref, bet_ref, mu_ref, var_ref, out_ref, obf_ref):
    t2 = 2.0 * (q0_ref[0] + q1_ref[0]) - v_ref[...]
    h = part_ref[...] + jnp.dot(t2, w2_ref[...], preferred_element_type=_f32)
    h = jnp.maximum(h + b_ref[...], 0.0)
    scale = gam_ref[...] * lax.rsqrt(var_ref[...] + 1e-5)
    hn = (h - mu_ref[...]) * scale + bet_ref[...]
    out_ref[...] = hn
    obf_ref[...] = _bf16_pack(hn)


def _tc_layer1(q, v, part, w2, b, gam, bet, mu, var):
    vec = pl.BlockSpec((1, F), lambda i: (0, 0))
    return pl.pallas_call(
        _tc_layer1_body,
        grid=(GRID,),
        in_specs=[
            pl.BlockSpec((1, BN_ROWS, F), lambda i: (0, i, 0)),
            pl.BlockSpec((1, BN_ROWS, F), lambda i: (1, i, 0)),
            pl.BlockSpec((BN_ROWS, F), lambda i: (i, 0)),
            pl.BlockSpec((BN_ROWS, F), lambda i: (i, 0)),
            pl.BlockSpec((F, F), lambda i: (0, 0)),
            vec, vec, vec, vec, vec,
        ],
        out_specs=(pl.BlockSpec((BN_ROWS, F), lambda i: (i, 0)),
                   pl.BlockSpec((BN_ROWS, F), lambda i: (i, 0))),
        out_shape=(jax.ShapeDtypeStruct((NP, F), _f32),
                   jax.ShapeDtypeStruct((NP, F), jnp.bfloat16)),
    )(q, q, v, part, w2, b.reshape(1, F), gam.reshape(1, F),
      bet.reshape(1, F), mu.reshape(1, F), var.reshape(1, F))


def _tc_layer2_body(q0_ref, q1_ref, v_ref, part_ref, w2_ref, b_ref,
                    lw_ref, lb_ref, out_ref):
    t2 = 2.0 * (q0_ref[0] + q1_ref[0]) - v_ref[...]
    h = part_ref[...] + jnp.dot(t2, w2_ref[...], preferred_element_type=_f32)
    h = jnp.maximum(h + b_ref[...], 0.0)
    out_ref[...] = jnp.dot(h, lw_ref[...], preferred_element_type=_f32) + lb_ref[...]


def _tc_layer2(q, v, part, w2, b, lw, lb):
    return pl.pallas_call(
        _tc_layer2_body,
        grid=(GRID,),
        in_specs=[
            pl.BlockSpec((1, BN_ROWS, F), lambda i: (0, i, 0)),
            pl.BlockSpec((1, BN_ROWS, F), lambda i: (1, i, 0)),
            pl.BlockSpec((BN_ROWS, F), lambda i: (i, 0)),
            pl.BlockSpec((BN_ROWS, F), lambda i: (i, 0)),
            pl.BlockSpec((F, F), lambda i: (0, 0)),
            pl.BlockSpec((1, F), lambda i: (0, 0)),
            pl.BlockSpec((F, OUT_F), lambda i: (0, 0)),
            pl.BlockSpec((1, OUT_F), lambda i: (0, 0)),
        ],
        out_specs=pl.BlockSpec((BN_ROWS, OUT_F), lambda i: (i, 0)),
        out_shape=jax.ShapeDtypeStruct((NP, OUT_F), _f32),
    )(q, q, v, part, w2, b.reshape(1, F), lw, lb.reshape(1, OUT_F))


def kernel(x, edge_index, edge_weight, W1, b1, bn_gamma, bn_beta,
           bn_mean, bn_var, W2, b2, lin_w, lin_b):
    pad = EPAD - E
    pad_idx = N + (jnp.arange(pad, dtype=_i32) % (NP - N))
    src = jnp.concatenate([edge_index[0], pad_idx]).reshape(NW, NB, EB)
    dst = jnp.concatenate([edge_index[1], pad_idx]).reshape(NW, NB, EB)
    ew = jnp.pad(edge_weight, (0, pad)).reshape(NW, NB, EB)
    src4 = src.reshape(NW, NCH, CB, EB)
    dst4 = dst.reshape(NW, NCH, CB, EB)
    xp = jnp.pad(x, ((0, NP - N), (0, 0)))

    wn = _wnorm(src, dst, ew)

    wn4 = wn.reshape(NW, NCH, CB, EB)
    xbf = _tc_pack(xp)
    p = _lhat(xbf, src4, dst4, wn4)
    tx1, tx1bf = _tc_add2(p)
    q = _lhat(tx1bf, src4, dst4, wn4)
    part1 = _tc_part(xp, tx1, W1[0], W1[1])
    h, hbf = _tc_layer1(q, xp, part1, W1[2], b1, bn_gamma, bn_beta, bn_mean, bn_var)

    r = _lhat(hbf, src4, dst4, wn4)
    tx1b, tx1bbf = _tc_add2(r)
    sP = _lhat(tx1bbf, src4, dst4, wn4)
    part2 = _tc_part(h, tx1b, W2[0], W2[1])
    out = _tc_layer2(sP, h, part2, W2[2], b2, lin_w, lin_b)

    return out[:N]


# speedup vs baseline: 2.7443x; 2.7443x over previous
import functools

import jax
import jax.numpy as jnp
from jax import lax
from jax.experimental import pallas as pl
from jax.experimental.pallas import tpu as pltpu
from jax.experimental.pallas import tpu_sc as plsc

N = 10000
NP = 10240
E = 320000
F = 128
OUT_F = 64

NC = 2
NS = 16
L = 16
NW = NC * NS
EPT = E // NW
EPS = E // NS
RPT = NP // NS

EB = 128
NB = 80
CB = 40
NCH = NB // CB
EPAD = NW * NB * EB
ZR = 128

_SC_PARAMS = pltpu.CompilerParams(needs_layout_passes=False)

_f32 = jnp.float32
_i32 = jnp.int32


def _mesh():
    return plsc.VectorSubcoreMesh(
        core_axis_name="c", subcore_axis_name="s",
        num_cores=NC, num_subcores=NS)


def _rsqrt16(d):
    bits = plsc.bitcast(d, _i32)
    y = plsc.bitcast(jnp.int32(0x5F3759DF) - (bits >> 1), _f32)
    for _ in range(3):
        y = y * (1.5 - 0.5 * d * y * y)
    return jnp.where(d > 0.0, y, 0.0)


@functools.partial(
    pl.kernel,
    out_type=jax.ShapeDtypeStruct((NW, NB, EB), _f32),
    mesh=_mesh(),
    scratch_types=[
        pltpu.VMEM_SHARED((NS, NP), _f32),
        pltpu.VMEM_SHARED((NP,), _f32),
        pltpu.VMEM((NP,), _f32),
        pltpu.VMEM((NP,), _f32),
        pltpu.VMEM((NS, RPT), _f32),
        pltpu.VMEM((NB, EB), _i32),
        pltpu.VMEM((NB, EB), _i32),
        pltpu.VMEM((NB, EB), _f32),
        pltpu.VMEM((NB, EB), _f32),
    ],
    compiler_params=_SC_PARAMS,
)
def _wnorm(src_hbm, dst_hbm, ew_hbm, wn_hbm,
           deg_sh, dinv_sh, deg_v, dinv_v, sum_v, sv_v, dv_v, ev_v, wn_v):
    c = lax.axis_index("c")
    s = lax.axis_index("s")
    wid = c * NS + s

    def zero_deg(i, _):
        deg_v[pl.ds(i * L, L)] = jnp.zeros((L,), _f32)
        return 0
    lax.fori_loop(0, NP // L, zero_deg, 0)

    for half in range(2):
        chunk = s * 2 + half
        pltpu.sync_copy(src_hbm.at[chunk], sv_v)
        pltpu.sync_copy(ew_hbm.at[chunk], ev_v)

        def deg_batch(g, _):
            def inner(j, _):
                idx = sv_v[g, pl.ds(j * L, L)]
                val = ev_v[g, pl.ds(j * L, L)]
                plsc.addupdate_scatter(deg_v, [idx], val)
                return 0
            lax.fori_loop(0, EB // L, inner, 0)
            return 0
        lax.fori_loop(0, NB, deg_batch, 0)

    pltpu.sync_copy(deg_v, deg_sh.at[s])
    plsc.subcore_barrier()

    pltpu.sync_copy(deg_sh.at[:, pl.ds(s * RPT, RPT)], sum_v)

    def dinv_blk(j, _):
        d = jnp.zeros((L,), _f32)
        for r in range(NS):
            d = d + sum_v[r, pl.ds(j * L, L)]
        dinv_v[pl.ds(s * RPT + j * L, L)] = _rsqrt16(d)
        return 0
    lax.fori_loop(0, RPT // L, dinv_blk, 0)

    pltpu.sync_copy(dinv_v.at[pl.ds(s * RPT, RPT)],
                    dinv_sh.at[pl.ds(s * RPT, RPT)])
    plsc.subcore_barrier()
    pltpu.sync_copy(dinv_sh, dinv_v)

    pltpu.sync_copy(src_hbm.at[wid], sv_v)
    pltpu.sync_copy(dst_hbm.at[wid], dv_v)
    pltpu.sync_copy(ew_hbm.at[wid], ev_v)

    def wn_batch(g, _):
        def inner(j, _):
            si = sv_v[g, pl.ds(j * L, L)]
            di = dv_v[g, pl.ds(j * L, L)]
            ew = ev_v[g, pl.ds(j * L, L)]
            a = plsc.load_gather(dinv_v, [si])
            b = plsc.load_gather(dinv_v, [di])
            wn_v[g, pl.ds(j * L, L)] = -(a * ew * b)
            return 0
        lax.fori_loop(0, EB // L, inner, 0)
        return 0
    lax.fori_loop(0, NB, wn_batch, 0)
    pltpu.sync_copy(wn_v, wn_hbm.at[wid])


@functools.partial(
    pl.kernel,
    out_type=jax.ShapeDtypeStruct((NC, NP, F), _f32),
    mesh=_mesh(),
    scratch_types=[
        pltpu.VMEM_SHARED((NP, F), _f32),
        pltpu.VMEM((CB, EB), _i32),
        pltpu.VMEM((CB, EB), _i32),
        pltpu.VMEM((CB, EB), _f32),
        pltpu.VMEM((EB, F), _f32),
        pltpu.VMEM((EB, F), _f32),
        pltpu.SemaphoreType.DMA,
        pltpu.SemaphoreType.DMA,
    ],
    compiler_params=_SC_PARAMS,
)
def _lhat(v_hbm, src_hbm, dst_hbm, wn_hbm, p_hbm,
          acc_sh, sv_v, dv_v, wv_v, rows0, rows1, sg0, sg1):
    c = lax.axis_index("c")
    s = lax.axis_index("s")
    wid = c * NS + s

    def zero_st(i, _):
        rows0[i // (F // L), pl.ds((i % (F // L)) * L, L)] = jnp.zeros((L,), _f32)
        return 0
    lax.fori_loop(0, ZR * (F // L), zero_st, 0)
    for t in range(RPT // ZR):
        pltpu.sync_copy(rows0, acc_sh.at[pl.ds(s * RPT + t * ZR, ZR)])
    plsc.subcore_barrier()

    def scale(rows, lg):
        def sc(j, _):
            w16 = wv_v[lg, pl.ds(j * L, L)]
            for bb in range(L):
                b = j * L + bb
                w = w16[bb]
                for f in range(F // L):
                    rows[b, pl.ds(f * L, L)] = rows[b, pl.ds(f * L, L)] * w
            return 0
        lax.fori_loop(0, EB // L, sc, 0)

    for k in range(NCH):
        pltpu.sync_copy(src_hbm.at[wid, k], sv_v)
        pltpu.sync_copy(dst_hbm.at[wid, k], dv_v)
        pltpu.sync_copy(wn_hbm.at[wid, k], wv_v)

        pltpu.async_copy(v_hbm.at[sv_v.at[0]], rows0, sg0)
        pltpu.async_copy(v_hbm.at[sv_v.at[1]], rows1, sg1)

        def pipe(gg, _):
            g0 = 2 * gg
            g1 = 2 * gg + 1
            pltpu.make_async_copy(v_hbm.at[sv_v.at[g0]], rows0, sg0).wait()
            scale(rows0, g0)
            pltpu.sync_copy(rows0, acc_sh.at[dv_v.at[g0]], add=True)
            pltpu.async_copy(v_hbm.at[sv_v.at[g0 + 2]], rows0, sg0)
            pltpu.make_async_copy(v_hbm.at[sv_v.at[g1]], rows1, sg1).wait()
            scale(rows1, g1)
            pltpu.sync_copy(rows1, acc_sh.at[dv_v.at[g1]], add=True)
            pltpu.async_copy(v_hbm.at[sv_v.at[g1 + 2]], rows1, sg1)
            return 0
        lax.fori_loop(0, CB // 2 - 1, pipe, 0)

        pltpu.make_async_copy(v_hbm.at[sv_v.at[CB - 2]], rows0, sg0).wait()
        scale(rows0, CB - 2)
        pltpu.sync_copy(rows0, acc_sh.at[dv_v.at[CB - 2]], add=True)
        pltpu.make_async_copy(v_hbm.at[sv_v.at[CB - 1]], rows1, sg1).wait()
        scale(rows1, CB - 1)
        pltpu.sync_copy(rows1, acc_sh.at[dv_v.at[CB - 1]], add=True)

    plsc.subcore_barrier()
    for t in range(RPT // ZR):
        pltpu.sync_copy(acc_sh.at[pl.ds(s * RPT + t * ZR, ZR)],
                        p_hbm.at[c, pl.ds(s * RPT + t * ZR, ZR)])


BN_ROWS = 256
GRID = NP // BN_ROWS


def _tc_add2_body(p0_ref, p1_ref, tx1_ref):
    tx1_ref[...] = p0_ref[0] + p1_ref[0]


def _tc_add2(p):
    return pl.pallas_call(
        _tc_add2_body,
        grid=(GRID,),
        in_specs=[
            pl.BlockSpec((1, BN_ROWS, F), lambda i: (0, i, 0)),
            pl.BlockSpec((1, BN_ROWS, F), lambda i: (1, i, 0)),
        ],
        out_specs=pl.BlockSpec((BN_ROWS, F), lambda i: (i, 0)),
        out_shape=jax.ShapeDtypeStruct((NP, F), _f32),
    )(p, p)


def _tc_part_body(v_ref, tx1_ref, w0_ref, w1_ref, part_ref):
    part_ref[...] = (
        jnp.dot(v_ref[...], w0_ref[...], preferred_element_type=_f32)
        + jnp.dot(tx1_ref[...], w1_ref[...], preferred_element_type=_f32))


def _tc_part(v, tx1, w0, w1):
    return pl.pallas_call(
        _tc_part_body,
        grid=(GRID,),
        in_specs=[
            pl.BlockSpec((BN_ROWS, F), lambda i: (i, 0)),
            pl.BlockSpec((BN_ROWS, F), lambda i: (i, 0)),
            pl.BlockSpec((F, F), lambda i: (0, 0)),
            pl.BlockSpec((F, F), lambda i: (0, 0)),
        ],
        out_specs=pl.BlockSpec((BN_ROWS, F), lambda i: (i, 0)),
        out_shape=jax.ShapeDtypeStruct((NP, F), _f32),
    )(v, tx1, w0, w1)


def _tc_layer1_body(q0_ref, q1_ref, v_ref, part_ref, w2_ref, b_ref,
                    gam_ref, bet_ref, mu_ref, var_ref, out_ref):
    t2 = 2.0 * (q0_ref[0] + q1_ref[0]) - v_ref[...]
    h = part_ref[...] + jnp.dot(t2, w2_ref[...], preferred_element_type=_f32)
    h = jnp.maximum(h + b_ref[...], 0.0)
    scale = gam_ref[...] * lax.rsqrt(var_ref[...] + 1e-5)
    out_ref[...] = (h - mu_ref[...]) * scale + bet_ref[...]


def _tc_layer1(q, v, part, w2, b, gam, bet, mu, var):
    vec = pl.BlockSpec((1, F), lambda i: (0, 0))
    return pl.pallas_call(
        _tc_layer1_body,
        grid=(GRID,),
        in_specs=[
            pl.BlockSpec((1, BN_ROWS, F), lambda i: (0, i, 0)),
            pl.BlockSpec((1, BN_ROWS, F), lambda i: (1, i, 0)),
            pl.BlockSpec((BN_ROWS, F), lambda i: (i, 0)),
            pl.BlockSpec((BN_ROWS, F), lambda i: (i, 0)),
            pl.BlockSpec((F, F), lambda i: (0, 0)),
            vec, vec, vec, vec, vec,
        ],
        out_specs=pl.BlockSpec((BN_ROWS, F), lambda i: (i, 0)),
        out_shape=jax.ShapeDtypeStruct((NP, F), _f32),
    )(q, q, v, part, w2, b.reshape(1, F), gam.reshape(1, F),
      bet.reshape(1, F), mu.reshape(1, F), var.reshape(1, F))


def _tc_layer2_body(q0_ref, q1_ref, v_ref, part_ref, w2_ref, b_ref,
                    lw_ref, lb_ref, out_ref):
    t2 = 2.0 * (q0_ref[0] + q1_ref[0]) - v_ref[...]
    h = part_ref[...] + jnp.dot(t2, w2_ref[...], preferred_element_type=_f32)
    h = jnp.maximum(h + b_ref[...], 0.0)
    out_ref[...] = jnp.dot(h, lw_ref[...], preferred_element_type=_f32) + lb_ref[...]


def _tc_layer2(q, v, part, w2, b, lw, lb):
    return pl.pallas_call(
        _tc_layer2_body,
        grid=(GRID,),
        in_specs=[
            pl.BlockSpec((1, BN_ROWS, F), lambda i: (0, i, 0)),
            pl.BlockSpec((1, BN_ROWS, F), lambda i: (1, i, 0)),
            pl.BlockSpec((BN_ROWS, F), lambda i: (i, 0)),
            pl.BlockSpec((BN_ROWS, F), lambda i: (i, 0)),
            pl.BlockSpec((F, F), lambda i: (0, 0)),
            pl.BlockSpec((1, F), lambda i: (0, 0)),
            pl.BlockSpec((F, OUT_F), lambda i: (0, 0)),
            pl.BlockSpec((1, OUT_F), lambda i: (0, 0)),
        ],
        out_specs=pl.BlockSpec((BN_ROWS, OUT_F), lambda i: (i, 0)),
        out_shape=jax.ShapeDtypeStruct((NP, OUT_F), _f32),
    )(q, q, v, part, w2, b.reshape(1, F), lw, lb.reshape(1, OUT_F))


def kernel(x, edge_index, edge_weight, W1, b1, bn_gamma, bn_beta,
           bn_mean, bn_var, W2, b2, lin_w, lin_b):
    pad = EPAD - E
    pad_idx = N + (jnp.arange(pad, dtype=_i32) % (NP - N))
    src = jnp.concatenate([edge_index[0], pad_idx]).reshape(NW, NB, EB)
    dst = jnp.concatenate([edge_index[1], pad_idx]).reshape(NW, NB, EB)
    ew = jnp.pad(edge_weight, (0, pad)).reshape(NW, NB, EB)
    src4 = src.reshape(NW, NCH, CB, EB)
    dst4 = dst.reshape(NW, NCH, CB, EB)
    xp = jnp.pad(x, ((0, NP - N), (0, 0)))

    wn = _wnorm(src, dst, ew)

    wn4 = wn.reshape(NW, NCH, CB, EB)
    p = _lhat(xp, src4, dst4, wn4)
    tx1 = _tc_add2(p)
    q = _lhat(tx1, src4, dst4, wn4)
    part1 = _tc_part(xp, tx1, W1[0], W1[1])
    h = _tc_layer1(q, xp, part1, W1[2], b1, bn_gamma, bn_beta, bn_mean, bn_var)

    r = _lhat(h, src4, dst4, wn4)
    tx1b = _tc_add2(r)
    sP = _lhat(tx1b, src4, dst4, wn4)
    part2 = _tc_part(h, tx1b, W2[0], W2[1])
    out = _tc_layer2(sP, h, part2, W2[2], b2, lin_w, lin_b)

    return out[:N]
